# Initial kernel scaffold; baseline (speedup 1.0000x reference)
import jax
import jax.numpy as jnp
from jax.experimental import pallas as pl


def kernel(x, Wg, W1, b1, W2, b2):
    raise NotImplementedError("write your pallas kernel here")


# speedup vs baseline: 1.2233x; 1.2233x over previous
import math

import jax
import jax.numpy as jnp
from jax.experimental import pallas as pl
from jax.experimental.pallas import tpu as pltpu

N_EMBD = 512
NUM_EXPERTS = 4
TOP_K = 2
D_FF = 4 * N_EMBD
_GELU_C = math.sqrt(2.0 / math.pi)


def _gelu(h):
    return 0.5 * h * (1.0 + jnp.tanh(_GELU_C * (h + 0.044715 * h ** 3)))


def _moe_body(x_ref, Wg_ref, W1_ref, b1_ref, W2_ref, b2_ref,
              out_ref, loss_ref, gsum_ref):
    i = pl.program_id(0)
    nb = pl.num_programs(0)
    xb = x_ref[...]
    g = jnp.dot(xb, Wg_ref[...], preferred_element_type=jnp.float32)

    @pl.when(i == 0)
    def _():
        gsum_ref[0, 0] = 0.0

    gsum_ref[0, 0] += jnp.sum(g)

    col = jax.lax.broadcasted_iota(jnp.int32, g.shape, 1)
    m1 = jnp.max(g, axis=1, keepdims=True)
    e1 = jnp.argmax(g, axis=1)
    gm = jnp.where(col == e1[:, None], -jnp.inf, g)
    m2 = jnp.max(gm, axis=1, keepdims=True)
    e2 = jnp.argmax(gm, axis=1)
    t = jnp.exp(m2 - m1)
    w1 = (1.0 / (1.0 + t))[:, 0]
    w2 = 1.0 - w1

    acc = jnp.zeros(out_ref.shape, dtype=jnp.float32)
    for k in range(NUM_EXPERTS):
        coeff = w1 * (e1 == k).astype(jnp.float32) + \
                w2 * (e2 == k).astype(jnp.float32)
        h = _gelu(jnp.dot(xb, W1_ref[k], preferred_element_type=jnp.float32)
                  + b1_ref[k][None, :])
        y = jnp.dot(h, W2_ref[k], preferred_element_type=jnp.float32) \
            + b2_ref[k][None, :]
        acc = acc + coeff[:, None] * y
    out_ref[...] = acc

    @pl.when(i == nb - 1)
    def _():
        s = gsum_ref[0, 0] / jnp.float32(nb * out_ref.shape[0] * NUM_EXPERTS)
        loss_ref[...] = jnp.broadcast_to(s * jnp.log(s + 0.1), (1, 1))


def kernel(x, Wg, W1, b1, W2, b2):
    T = x.shape[0] * x.shape[1]
    flat = x.reshape(T, N_EMBD)
    BT = 512
    nb = T // BT
    out, loss = pl.pallas_call(
        _moe_body,
        grid=(nb,),
        in_specs=[
            pl.BlockSpec((BT, N_EMBD), lambda i: (i, 0)),
            pl.BlockSpec((N_EMBD, NUM_EXPERTS), lambda i: (0, 0)),
            pl.BlockSpec((NUM_EXPERTS, N_EMBD, D_FF), lambda i: (0, 0, 0)),
            pl.BlockSpec((NUM_EXPERTS, D_FF), lambda i: (0, 0)),
            pl.BlockSpec((NUM_EXPERTS, D_FF, N_EMBD), lambda i: (0, 0, 0)),
            pl.BlockSpec((NUM_EXPERTS, N_EMBD), lambda i: (0, 0)),
        ],
        out_specs=[
            pl.BlockSpec((BT, N_EMBD), lambda i: (i, 0)),
            pl.BlockSpec((1, 1), lambda i: (0, 0)),
        ],
        out_shape=[
            jax.ShapeDtypeStruct((T, N_EMBD), jnp.float32),
            jax.ShapeDtypeStruct((1, 1), jnp.float32),
        ],
        scratch_shapes=[pltpu.SMEM((1, 1), jnp.float32)],
    )(flat, Wg, W1, b1, W2, b2)
    return out.reshape(x.shape), loss[0, 0]
